# Initial kernel scaffold; baseline (speedup 1.0000x reference)
import jax
import jax.numpy as jnp
from jax.experimental import pallas as pl


def kernel(X, stem_w, stem_gamma, stem_beta, gate_w0, gate_w1, exp_w0, exp_w1, gn_g0, gn_b0, gn_g1, gn_b1, head_w1, head_b1, head_w2, head_b2):
    raise NotImplementedError("write your pallas kernel here")


# speedup vs baseline: 7.4862x; 7.4862x over previous
import functools

import jax
import jax.numpy as jnp
import numpy as np
from jax.experimental import pallas as pl
from jax.experimental.pallas import tpu as pltpu

B, CIN, H, W = 32, 3, 224, 224
PATCH = 16
E = 8
F = 4
CF = 2 + 4 * F
CH = 8
NUM_CLASSES = 1000
GROUPS = 4
HP = H // PATCH
WP = W // PATCH
P = HP * WP
NTOK = B * P
CAP = int(np.ceil(NTOK / E * 2.0))
PP = 256


def _pos_const():
    ys = (np.arange(H) + 0.5) / H
    xs = (np.arange(W) + 0.5) / W
    yy, xx = np.meshgrid(ys, xs, indexing='ij')
    feats = [yy, xx]
    for f in range(F):
        wf = (2.0 ** f) * np.pi
        feats += [np.sin(wf * yy), np.cos(wf * yy), np.sin(wf * xx), np.cos(wf * xx)]
    pos = np.stack(feats, 0).astype(np.float32)
    return pos.reshape(CF, HP, PATCH, WP, PATCH).transpose(1, 3, 0, 2, 4).reshape(P, CF, PATCH, PATCH)


_POS = _pos_const()
_GF = np.zeros((PP, 128), np.float32)
_GF[:P, :CF] = _POS.mean(axis=(2, 3), dtype=np.float64).astype(np.float32)
_TRI = np.tril(np.ones((PP, PP), np.float32))

_M1 = (np.arange(B)[:, None] == (np.arange(B * PATCH)[None, :] // PATCH)).astype(np.float32)
_GSUM = (np.arange(128)[:, None] // 32 == np.arange(128)[None, :] // 32).astype(np.float32)
_P16 = (np.arange(128)[:, None] // 16 == np.arange(128)[None, :]).astype(np.float32)
_GC = np.zeros((128, 128), np.float32)
for _c in range(CH):
    for _c2 in range(CH):
        if _c // 2 == _c2 // 2:
            _GC[_c, _c2] = 1.0


def _routing_kernel(lg_ref, tri_ref, ex_ref, sc_ref):
    tri = tri_ref[...]
    lane = jax.lax.broadcasted_iota(jnp.int32, (PP, 128), 1)
    for l in range(2):
        lm = lg_ref[l]
        mx = jnp.max(lm, axis=1, keepdims=True)
        ex_exp = jnp.exp(lm - mx)
        probs = ex_exp / jnp.sum(ex_exp, axis=1, keepdims=True)
        gate = jnp.max(probs, axis=1, keepdims=True)
        eidx = jnp.argmax(lm, axis=1)
        rowmask = jax.lax.broadcasted_iota(jnp.int32, (PP, 1), 0) < P
        oh = jnp.where((lane == eidx[:, None]) & rowmask, 1.0, 0.0)
        cnt = jnp.sum(oh, axis=0, keepdims=True)
        ranks = jnp.dot(tri, oh, preferred_element_type=jnp.float32, precision=jax.lax.Precision.HIGHEST)
        rank_p = jnp.sum(ranks * oh, axis=1, keepdims=True) - 1.0
        cnt_p = jnp.sum(cnt * oh, axis=1, keepdims=True)
        bcol = lane.astype(jnp.float32)
        kept = ((cnt_p * bcol + rank_p) < CAP).astype(jnp.float32)
        ex_ref[l] = jnp.broadcast_to(eidx[:, None], (PP, 128))
        sc_ref[l] = gate * kept


def _stem_kernel(x_ref, w_ref, g_ref, b_ref, y_ref):
    x = x_ref[0]
    acc = jnp.zeros((CH, H, W), jnp.float32)
    bf = jnp.bfloat16
    for ci in range(CIN):
        ap = jnp.pad(x[ci], 1).astype(bf).astype(jnp.float32)
        for ky in range(3):
            for kx in range(3):
                sh = ap[ky:ky + H, kx:kx + W]
                wv = w_ref[:, ci * 9 + ky * 3 + kx].astype(bf).astype(jnp.float32)
                acc = acc + wv[:, None, None] * sh[None, :, :]
    acc = acc * g_ref[:, 0][:, None, None] + b_ref[:, 0][:, None, None]
    y_ref[0] = acc * jax.nn.sigmoid(acc)


def _moe_kernel(ex0_ref, ex1_ref, t_ref, wb_ref, rs0_ref, rs1_ref,
                m1_ref, m1t_ref, g_ref, p16_ref, gb0_ref,
                sums_ref, sumsq_ref, comp1_ref, comp2_ref):
    i = pl.program_id(0)
    x = t_ref[0]
    m1 = m1_ref[...]
    m1t = m1t_ref[...]
    ymod = jax.lax.broadcasted_iota(jnp.int32, (B * PATCH, 1), 0) % PATCH

    def conv(z, l, e):
        w3 = wb_ref[l, e]
        up = jnp.where(ymod == PATCH - 1, 0.0, jnp.roll(z, -1, axis=0))
        dn = jnp.where(ymod == 0, 0.0, jnp.roll(z, 1, axis=0))
        bf = jnp.bfloat16
        return (jnp.dot(dn.astype(bf), w3[0].astype(bf), preferred_element_type=jnp.float32)
                + jnp.dot(z.astype(bf), w3[1].astype(bf), preferred_element_type=jnp.float32)
                + jnp.dot(up.astype(bf), w3[2].astype(bf), preferred_element_type=jnp.float32))

    e0 = ex0_ref[i]
    y = conv(x, 0, e0)
    y = y * jax.nn.sigmoid(y)
    y = y * rs0_ref[0]
    s1 = jnp.dot(m1, y, preferred_element_type=jnp.float32, precision=jax.lax.Precision.HIGHEST)
    s2 = jnp.dot(m1, y * y, preferred_element_type=jnp.float32, precision=jax.lax.Precision.HIGHEST)
    tg = jnp.dot(s1, g_ref[...], preferred_element_type=jnp.float32, precision=jax.lax.Precision.HIGHEST)
    t2g = jnp.dot(s2, g_ref[...], preferred_element_type=jnp.float32, precision=jax.lax.Precision.HIGHEST)
    mean = tg * (1.0 / 512.0)
    var = t2g * (1.0 / 512.0) - mean * mean
    inv = jax.lax.rsqrt(var + 1e-5)
    gam = gb0_ref[0:1, :]
    bet = gb0_ref[1:2, :]
    a = inv * gam
    c = bet - mean * a
    z = y * jnp.dot(m1t, a, preferred_element_type=jnp.float32, precision=jax.lax.Precision.HIGHEST) \
        + jnp.dot(m1t, c, preferred_element_type=jnp.float32, precision=jax.lax.Precision.HIGHEST)
    e1 = ex1_ref[i]
    y2 = conv(z, 1, e1)
    y2 = y2 * jax.nn.sigmoid(y2)
    y2 = y2 * rs1_ref[0]
    cs = jnp.dot(jnp.dot(m1, y2, preferred_element_type=jnp.float32, precision=jax.lax.Precision.HIGHEST),
                 p16_ref[...], preferred_element_type=jnp.float32, precision=jax.lax.Precision.HIGHEST)
    cs2 = jnp.dot(jnp.dot(m1, y2 * y2, preferred_element_type=jnp.float32, precision=jax.lax.Precision.HIGHEST),
                  p16_ref[...], preferred_element_type=jnp.float32, precision=jax.lax.Precision.HIGHEST)

    @pl.when(i == 0)
    def _():
        sums_ref[...] = jnp.zeros_like(sums_ref)
        sumsq_ref[...] = jnp.zeros_like(sumsq_ref)
        comp1_ref[...] = jnp.zeros_like(comp1_ref)
        comp2_ref[...] = jnp.zeros_like(comp2_ref)

    for inc, acc_ref, comp_ref in ((cs, sums_ref, comp1_ref),
                                   (cs2, sumsq_ref, comp2_ref)):
        yk = inc - comp_ref[...]
        s_old = acc_ref[...]
        t = s_old + yk
        comp_ref[...] = (t - s_old) - yk
        acc_ref[...] = t


def _head_kernel(sums_ref, sumsq_ref, gc_ref, gb1_ref, w1_ref, b1_ref,
                 w2_ref, b2_ref, out_ref):
    hw = float(H * W)
    sums = sums_ref[...]
    sumsq = sumsq_ref[...]
    m = jnp.dot(sums, gc_ref[...], preferred_element_type=jnp.float32, precision=jax.lax.Precision.HIGHEST) * (1.0 / (2.0 * hw))
    ex2 = jnp.dot(sumsq, gc_ref[...], preferred_element_type=jnp.float32, precision=jax.lax.Precision.HIGHEST) * (1.0 / (2.0 * hw))
    var = ex2 - m * m
    inv = jax.lax.rsqrt(var + 1e-5)
    pooled = (sums * (1.0 / hw) - m) * inv * gb1_ref[0:1, :] + gb1_ref[1:2, :]
    h = jnp.dot(pooled, w1_ref[...], preferred_element_type=jnp.float32, precision=jax.lax.Precision.HIGHEST) + b1_ref[...]
    h = jax.nn.gelu(h)
    out_ref[...] = jnp.dot(h, w2_ref[...], preferred_element_type=jnp.float32, precision=jax.lax.Precision.HIGHEST) + b2_ref[...]


def _band_weights(ew):
    xi = np.arange(PATCH)[:, None]
    xo = np.arange(PATCH)[None, :]
    kxm = xi - xo + 1
    valid = jnp.asarray((kxm >= 0) & (kxm <= 2), jnp.float32)
    kxc = jnp.asarray(np.clip(kxm, 0, 2))
    w6 = ew[:, :, :, :, kxc]
    w6 = w6 * valid[None, None, None, None]
    w6 = jnp.transpose(w6, (0, 3, 2, 4, 1, 5))
    return w6.reshape(E, 3, 128, 128)


def kernel(X, stem_w, stem_gamma, stem_beta, gate_w0, gate_w1, exp_w0, exp_w1,
           gn_g0, gn_b0, gn_g1, gn_b1, head_w1, head_b1, head_w2, head_b2):
    f32 = jnp.float32

    gfm = jnp.asarray(_POS).mean(axis=(2, 3))
    gate_feat = jnp.tile(gfm, (B, 1))
    lg0 = (gate_feat @ gate_w0)[:P]
    lg1 = (gate_feat @ gate_w1)[:P]
    lgp = jnp.full((2, PP, 128), -1e30, f32)
    lgp = lgp.at[0, :P, :E].set(lg0).at[1, :P, :E].set(lg1)
    ex_out, sc_out = pl.pallas_call(
        _routing_kernel,
        out_shape=(jax.ShapeDtypeStruct((2, PP, 128), jnp.int32),
                   jax.ShapeDtypeStruct((2, PP, 128), f32)),
    )(lgp, jnp.asarray(_TRI))
    ex0 = ex_out[0, :P, 0]
    ex1 = ex_out[1, :P, 0]
    rs0 = jnp.repeat(sc_out[0, :P, :B], PATCH, axis=1).reshape(P, B * PATCH, 1)
    rs1 = jnp.repeat(sc_out[1, :P, :B], PATCH, axis=1).reshape(P, B * PATCH, 1)

    w2 = stem_w.reshape(CH, CIN * 9)
    Y = pl.pallas_call(
        _stem_kernel,
        grid=(B,),
        in_specs=[
            pl.BlockSpec((1, CIN, H, W), lambda b: (b, 0, 0, 0)),
            pl.BlockSpec((CH, CIN * 9), lambda b: (0, 0)),
            pl.BlockSpec((CH, 1), lambda b: (0, 0)),
            pl.BlockSpec((CH, 1), lambda b: (0, 0)),
        ],
        out_specs=pl.BlockSpec((1, CH, H, W), lambda b: (b, 0, 0, 0)),
        out_shape=jax.ShapeDtypeStruct((B, CH, H, W), f32),
    )(X, w2, stem_gamma[:, None], stem_beta[:, None])

    T = Y.reshape(B, CH, HP, PATCH, WP, PATCH).transpose(2, 4, 0, 3, 1, 5) \
         .reshape(P, B * PATCH, CH * PATCH)

    wb = jnp.stack([_band_weights(exp_w0), _band_weights(exp_w1)])
    gb0 = jnp.stack([jnp.repeat(gn_g0, PATCH), jnp.repeat(gn_b0, PATCH)])
    grid_spec = pltpu.PrefetchScalarGridSpec(
        num_scalar_prefetch=2,
        grid=(P,),
        in_specs=[
            pl.BlockSpec((1, B * PATCH, 128), lambda i, e0, e1: (i, 0, 0)),
            pl.BlockSpec((2, E, 3, 128, 128), lambda i, e0, e1: (0, 0, 0, 0, 0)),
            pl.BlockSpec((1, B * PATCH, 1), lambda i, e0, e1: (i, 0, 0)),
            pl.BlockSpec((1, B * PATCH, 1), lambda i, e0, e1: (i, 0, 0)),
            pl.BlockSpec((B, B * PATCH), lambda i, e0, e1: (0, 0)),
            pl.BlockSpec((B * PATCH, B), lambda i, e0, e1: (0, 0)),
            pl.BlockSpec((128, 128), lambda i, e0, e1: (0, 0)),
            pl.BlockSpec((128, 128), lambda i, e0, e1: (0, 0)),
            pl.BlockSpec((2, 128), lambda i, e0, e1: (0, 0)),
        ],
        out_specs=tuple(pl.BlockSpec((B, 128), lambda i, e0, e1: (0, 0))
                        for _ in range(4)),
    )
    sums, sumsq, _, _ = pl.pallas_call(
        _moe_kernel,
        grid_spec=grid_spec,
        out_shape=tuple(jax.ShapeDtypeStruct((B, 128), f32) for _ in range(4)),
    )(ex0, ex1, T, wb, rs0, rs1,
      jnp.asarray(_M1), jnp.asarray(_M1.T), jnp.asarray(_GSUM), jnp.asarray(_P16), gb0)

    gb1 = jnp.zeros((2, 128), f32)
    gb1 = gb1.at[0, :CH].set(gn_g1).at[1, :CH].set(gn_b1)
    w1p = jnp.zeros((128, 128), f32).at[:CH, :4 * CH].set(head_w1)
    b1p = jnp.zeros((1, 128), f32).at[0, :4 * CH].set(head_b1)
    w2p = jnp.zeros((128, 1024), f32).at[:4 * CH, :NUM_CLASSES].set(head_w2)
    b2p = jnp.zeros((1, 1024), f32).at[0, :NUM_CLASSES].set(head_b2)
    out = pl.pallas_call(
        _head_kernel,
        out_shape=jax.ShapeDtypeStruct((B, 1024), f32),
    )(sums, sumsq, jnp.asarray(_GC), gb1, w1p, b1p, w2p, b2p)
    return out[:, :NUM_CLASSES]


# speedup vs baseline: 12.0535x; 1.6101x over previous
import functools

import jax
import jax.numpy as jnp
import numpy as np
from jax.experimental import pallas as pl
from jax.experimental.pallas import tpu as pltpu

B, CIN, H, W = 32, 3, 224, 224
PATCH = 16
E = 8
F = 4
CF = 2 + 4 * F
CH = 8
NUM_CLASSES = 1000
GROUPS = 4
HP = H // PATCH
WP = W // PATCH
P = HP * WP
NTOK = B * P
CAP = int(np.ceil(NTOK / E * 2.0))
PP = 256


def _pos_const():
    ys = (np.arange(H) + 0.5) / H
    xs = (np.arange(W) + 0.5) / W
    yy, xx = np.meshgrid(ys, xs, indexing='ij')
    feats = [yy, xx]
    for f in range(F):
        wf = (2.0 ** f) * np.pi
        feats += [np.sin(wf * yy), np.cos(wf * yy), np.sin(wf * xx), np.cos(wf * xx)]
    pos = np.stack(feats, 0).astype(np.float32)
    return pos.reshape(CF, HP, PATCH, WP, PATCH).transpose(1, 3, 0, 2, 4).reshape(P, CF, PATCH, PATCH)


_POS = _pos_const()
_GF = np.zeros((PP, 128), np.float32)
_GF[:P, :CF] = _POS.mean(axis=(2, 3), dtype=np.float64).astype(np.float32)
_TRI = np.tril(np.ones((PP, PP), np.float32))

_M1 = (np.arange(B)[:, None] == (np.arange(B * PATCH)[None, :] // PATCH)).astype(np.float32)
_GSUM = (np.arange(128)[:, None] // 32 == np.arange(128)[None, :] // 32).astype(np.float32)
_P16 = (np.arange(128)[:, None] // 16 == np.arange(128)[None, :]).astype(np.float32)
_GC = np.zeros((128, 128), np.float32)
for _c in range(CH):
    for _c2 in range(CH):
        if _c // 2 == _c2 // 2:
            _GC[_c, _c2] = 1.0


def _routing_kernel(lg_ref, tri_ref, ex_ref, sc_ref):
    tri = tri_ref[...]
    lane = jax.lax.broadcasted_iota(jnp.int32, (PP, 128), 1)
    for l in range(2):
        lm = lg_ref[l]
        mx = jnp.max(lm, axis=1, keepdims=True)
        ex_exp = jnp.exp(lm - mx)
        probs = ex_exp / jnp.sum(ex_exp, axis=1, keepdims=True)
        gate = jnp.max(probs, axis=1, keepdims=True)
        eidx = jnp.argmax(lm, axis=1)
        rowmask = jax.lax.broadcasted_iota(jnp.int32, (PP, 1), 0) < P
        oh = jnp.where((lane == eidx[:, None]) & rowmask, 1.0, 0.0)
        cnt = jnp.sum(oh, axis=0, keepdims=True)
        ranks = jnp.dot(tri, oh, preferred_element_type=jnp.float32, precision=jax.lax.Precision.HIGHEST)
        rank_p = jnp.sum(ranks * oh, axis=1, keepdims=True) - 1.0
        cnt_p = jnp.sum(cnt * oh, axis=1, keepdims=True)
        bcol = lane.astype(jnp.float32)
        kept = ((cnt_p * bcol + rank_p) < CAP).astype(jnp.float32)
        ex_ref[l] = jnp.broadcast_to(eidx[:, None], (PP, 128))
        sc_ref[l] = gate * kept


def _stem_kernel(xs_ref, wb_ref, gb_ref, out_ref):
    xs = xs_ref[0]
    ymod = jax.lax.broadcasted_iota(jnp.int32, (B * H, 1), 0) % H
    up = jnp.where(ymod == H - 1, 0.0, jnp.roll(xs, -1, axis=0))
    dn = jnp.where(ymod == 0, 0.0, jnp.roll(xs, 1, axis=0))
    bf = jnp.bfloat16
    acc = (jnp.dot(dn.astype(bf), wb_ref[0].astype(bf), preferred_element_type=jnp.float32)
           + jnp.dot(xs.astype(bf), wb_ref[1].astype(bf), preferred_element_type=jnp.float32)
           + jnp.dot(up.astype(bf), wb_ref[2].astype(bf), preferred_element_type=jnp.float32))
    acc = acc * gb_ref[0:1, :] + gb_ref[1:2, :]
    z = acc * jax.nn.sigmoid(acc)
    out_ref[0] = z.reshape(B, HP, PATCH, 128)


def _moe_kernel(ex0_ref, ex1_ref, t_ref, wb_ref, rs0_ref, rs1_ref,
                m1_ref, m1t_ref, g_ref, p16_ref, gb0_ref,
                sums_ref, sumsq_ref, comp1_ref, comp2_ref):
    i = pl.program_id(0)
    x = t_ref[0, :, 0].reshape(B * PATCH, 128)
    m1 = m1_ref[...]
    m1t = m1t_ref[...]
    ymod = jax.lax.broadcasted_iota(jnp.int32, (B * PATCH, 1), 0) % PATCH

    def conv(z, l, e):
        w3 = wb_ref[l, e]
        up = jnp.where(ymod == PATCH - 1, 0.0, jnp.roll(z, -1, axis=0))
        dn = jnp.where(ymod == 0, 0.0, jnp.roll(z, 1, axis=0))
        bf = jnp.bfloat16
        return (jnp.dot(dn.astype(bf), w3[0].astype(bf), preferred_element_type=jnp.float32)
                + jnp.dot(z.astype(bf), w3[1].astype(bf), preferred_element_type=jnp.float32)
                + jnp.dot(up.astype(bf), w3[2].astype(bf), preferred_element_type=jnp.float32))

    e0 = ex0_ref[i]
    y = conv(x, 0, e0)
    y = y * jax.nn.sigmoid(y)
    y = y * rs0_ref[0]
    s1 = jnp.dot(m1, y, preferred_element_type=jnp.float32, precision=jax.lax.Precision.HIGHEST)
    s2 = jnp.dot(m1, y * y, preferred_element_type=jnp.float32, precision=jax.lax.Precision.HIGHEST)
    tg = jnp.dot(s1, g_ref[...], preferred_element_type=jnp.float32, precision=jax.lax.Precision.HIGHEST)
    t2g = jnp.dot(s2, g_ref[...], preferred_element_type=jnp.float32, precision=jax.lax.Precision.HIGHEST)
    mean = tg * (1.0 / 512.0)
    var = t2g * (1.0 / 512.0) - mean * mean
    inv = jax.lax.rsqrt(var + 1e-5)
    gam = gb0_ref[0:1, :]
    bet = gb0_ref[1:2, :]
    a = inv * gam
    c = bet - mean * a
    z = y * jnp.dot(m1t, a, preferred_element_type=jnp.float32, precision=jax.lax.Precision.HIGHEST) \
        + jnp.dot(m1t, c, preferred_element_type=jnp.float32, precision=jax.lax.Precision.HIGHEST)
    e1 = ex1_ref[i]
    y2 = conv(z, 1, e1)
    y2 = y2 * jax.nn.sigmoid(y2)
    y2 = y2 * rs1_ref[0]
    cs = jnp.dot(jnp.dot(m1, y2, preferred_element_type=jnp.float32, precision=jax.lax.Precision.HIGHEST),
                 p16_ref[...], preferred_element_type=jnp.float32, precision=jax.lax.Precision.HIGHEST)
    cs2 = jnp.dot(jnp.dot(m1, y2 * y2, preferred_element_type=jnp.float32, precision=jax.lax.Precision.HIGHEST),
                  p16_ref[...], preferred_element_type=jnp.float32, precision=jax.lax.Precision.HIGHEST)

    @pl.when(i == 0)
    def _():
        sums_ref[...] = jnp.zeros_like(sums_ref)
        sumsq_ref[...] = jnp.zeros_like(sumsq_ref)
        comp1_ref[...] = jnp.zeros_like(comp1_ref)
        comp2_ref[...] = jnp.zeros_like(comp2_ref)

    for inc, acc_ref, comp_ref in ((cs, sums_ref, comp1_ref),
                                   (cs2, sumsq_ref, comp2_ref)):
        yk = inc - comp_ref[...]
        s_old = acc_ref[...]
        t = s_old + yk
        comp_ref[...] = (t - s_old) - yk
        acc_ref[...] = t


def _head_kernel(sums_ref, sumsq_ref, gc_ref, gb1_ref, w1_ref, b1_ref,
                 w2_ref, b2_ref, out_ref):
    hw = float(H * W)
    sums = sums_ref[...]
    sumsq = sumsq_ref[...]
    m = jnp.dot(sums, gc_ref[...], preferred_element_type=jnp.float32, precision=jax.lax.Precision.HIGHEST) * (1.0 / (2.0 * hw))
    ex2 = jnp.dot(sumsq, gc_ref[...], preferred_element_type=jnp.float32, precision=jax.lax.Precision.HIGHEST) * (1.0 / (2.0 * hw))
    var = ex2 - m * m
    inv = jax.lax.rsqrt(var + 1e-5)
    pooled = (sums * (1.0 / hw) - m) * inv * gb1_ref[0:1, :] + gb1_ref[1:2, :]
    h = jnp.dot(pooled, w1_ref[...], preferred_element_type=jnp.float32, precision=jax.lax.Precision.HIGHEST) + b1_ref[...]
    h = jax.nn.gelu(h)
    out_ref[...] = jnp.dot(h, w2_ref[...], preferred_element_type=jnp.float32, precision=jax.lax.Precision.HIGHEST) + b2_ref[...]


def _band_weights(ew):
    xi = np.arange(PATCH)[:, None]
    xo = np.arange(PATCH)[None, :]
    kxm = xi - xo + 1
    valid = jnp.asarray((kxm >= 0) & (kxm <= 2), jnp.float32)
    kxc = jnp.asarray(np.clip(kxm, 0, 2))
    w6 = ew[:, :, :, :, kxc]
    w6 = w6 * valid[None, None, None, None]
    w6 = jnp.transpose(w6, (0, 3, 2, 4, 1, 5))
    return w6.reshape(E, 3, 128, 128)


def kernel(X, stem_w, stem_gamma, stem_beta, gate_w0, gate_w1, exp_w0, exp_w1,
           gn_g0, gn_b0, gn_g1, gn_b1, head_w1, head_b1, head_w2, head_b2):
    f32 = jnp.float32

    gfm = jnp.asarray(_POS).mean(axis=(2, 3))
    gate_feat = jnp.tile(gfm, (B, 1))
    lg0 = (gate_feat @ gate_w0)[:P]
    lg1 = (gate_feat @ gate_w1)[:P]
    lgp = jnp.full((2, PP, 128), -1e30, f32)
    lgp = lgp.at[0, :P, :E].set(lg0).at[1, :P, :E].set(lg1)
    ex_out, sc_out = pl.pallas_call(
        _routing_kernel,
        out_shape=(jax.ShapeDtypeStruct((2, PP, 128), jnp.int32),
                   jax.ShapeDtypeStruct((2, PP, 128), f32)),
    )(lgp, jnp.asarray(_TRI))
    ex0 = ex_out[0, :P, 0]
    ex1 = ex_out[1, :P, 0]
    rs0 = jnp.repeat(sc_out[0, :P, :B], PATCH, axis=1).reshape(P, B * PATCH, 1)
    rs1 = jnp.repeat(sc_out[1, :P, :B], PATCH, axis=1).reshape(P, B * PATCH, 1)

    Xp = jnp.pad(X, ((0, 0), (0, 0), (0, 0), (1, 1)))
    Xs = jnp.stack([Xp[:, :, :, wp * PATCH:wp * PATCH + 18] for wp in range(WP)], 0)
    Xs = Xs.transpose(0, 1, 3, 2, 4).reshape(WP, B * H, CIN * 18)
    Xs = jnp.pad(Xs, ((0, 0), (0, 0), (0, 64 - CIN * 18)))
    cg, cig, dyg, kxg, xog = np.meshgrid(np.arange(CH), np.arange(CIN),
                                         np.arange(3), np.arange(3),
                                         np.arange(PATCH), indexing='ij')
    wsb = jnp.zeros((3, 64, 128), f32).at[
        dyg.ravel(), (cig * 18 + xog + kxg).ravel(), (cg * PATCH + xog).ravel()
    ].set(stem_w[cg.ravel(), cig.ravel(), dyg.ravel(), kxg.ravel()])
    gbs = jnp.stack([jnp.repeat(stem_gamma, PATCH), jnp.repeat(stem_beta, PATCH)])
    T = pl.pallas_call(
        _stem_kernel,
        grid=(WP,),
        in_specs=[
            pl.BlockSpec((1, B * H, 64), lambda w: (w, 0, 0)),
            pl.BlockSpec((3, 64, 128), lambda w: (0, 0, 0)),
            pl.BlockSpec((2, 128), lambda w: (0, 0)),
        ],
        out_specs=pl.BlockSpec((1, B, HP, PATCH, 128), lambda w: (w, 0, 0, 0, 0)),
        out_shape=jax.ShapeDtypeStruct((WP, B, HP, PATCH, 128), f32),
    )(Xs, wsb, gbs)

    wb = jnp.stack([_band_weights(exp_w0), _band_weights(exp_w1)])
    gb0 = jnp.stack([jnp.repeat(gn_g0, PATCH), jnp.repeat(gn_b0, PATCH)])
    grid_spec = pltpu.PrefetchScalarGridSpec(
        num_scalar_prefetch=2,
        grid=(P,),
        in_specs=[
            pl.BlockSpec((1, B, 1, PATCH, 128),
                         lambda i, e0, e1: (i % WP, 0, i // WP, 0, 0)),
            pl.BlockSpec((2, E, 3, 128, 128), lambda i, e0, e1: (0, 0, 0, 0, 0)),
            pl.BlockSpec((1, B * PATCH, 1), lambda i, e0, e1: (i, 0, 0)),
            pl.BlockSpec((1, B * PATCH, 1), lambda i, e0, e1: (i, 0, 0)),
            pl.BlockSpec((B, B * PATCH), lambda i, e0, e1: (0, 0)),
            pl.BlockSpec((B * PATCH, B), lambda i, e0, e1: (0, 0)),
            pl.BlockSpec((128, 128), lambda i, e0, e1: (0, 0)),
            pl.BlockSpec((128, 128), lambda i, e0, e1: (0, 0)),
            pl.BlockSpec((2, 128), lambda i, e0, e1: (0, 0)),
        ],
        out_specs=tuple(pl.BlockSpec((B, 128), lambda i, e0, e1: (0, 0))
                        for _ in range(4)),
    )
    sums, sumsq, _, _ = pl.pallas_call(
        _moe_kernel,
        grid_spec=grid_spec,
        out_shape=tuple(jax.ShapeDtypeStruct((B, 128), f32) for _ in range(4)),
    )(ex0, ex1, T, wb, rs0, rs1,
      jnp.asarray(_M1), jnp.asarray(_M1.T), jnp.asarray(_GSUM), jnp.asarray(_P16), gb0)

    gb1 = jnp.zeros((2, 128), f32)
    gb1 = gb1.at[0, :CH].set(gn_g1).at[1, :CH].set(gn_b1)
    w1p = jnp.zeros((128, 128), f32).at[:CH, :4 * CH].set(head_w1)
    b1p = jnp.zeros((1, 128), f32).at[0, :4 * CH].set(head_b1)
    w2p = jnp.zeros((128, 1024), f32).at[:4 * CH, :NUM_CLASSES].set(head_w2)
    b2p = jnp.zeros((1, 1024), f32).at[0, :NUM_CLASSES].set(head_b2)
    out = pl.pallas_call(
        _head_kernel,
        out_shape=jax.ShapeDtypeStruct((B, 1024), f32),
    )(sums, sumsq, jnp.asarray(_GC), gb1, w1p, b1p, w2p, b2p)
    return out[:, :NUM_CLASSES]


# speedup vs baseline: 12.8626x; 1.0671x over previous
import functools

import jax
import jax.numpy as jnp
import numpy as np
from jax.experimental import pallas as pl
from jax.experimental.pallas import tpu as pltpu

B, CIN, H, W = 32, 3, 224, 224
PATCH = 16
E = 8
F = 4
CF = 2 + 4 * F
CH = 8
NUM_CLASSES = 1000
GROUPS = 4
HP = H // PATCH
WP = W // PATCH
P = HP * WP
NTOK = B * P
CAP = int(np.ceil(NTOK / E * 2.0))
PP = 256


def _pos_const():
    ys = (np.arange(H) + 0.5) / H
    xs = (np.arange(W) + 0.5) / W
    yy, xx = np.meshgrid(ys, xs, indexing='ij')
    feats = [yy, xx]
    for f in range(F):
        wf = (2.0 ** f) * np.pi
        feats += [np.sin(wf * yy), np.cos(wf * yy), np.sin(wf * xx), np.cos(wf * xx)]
    pos = np.stack(feats, 0).astype(np.float32)
    return pos.reshape(CF, HP, PATCH, WP, PATCH).transpose(1, 3, 0, 2, 4).reshape(P, CF, PATCH, PATCH)


_POS = _pos_const()
_GF = np.zeros((PP, 128), np.float32)
_GF[:P, :CF] = _POS.mean(axis=(2, 3), dtype=np.float64).astype(np.float32)
_TRI = np.tril(np.ones((PP, PP), np.float32))

_M1 = (np.arange(B)[:, None] == (np.arange(B * PATCH)[None, :] // PATCH)).astype(np.float32)
_GSUM = (np.arange(128)[:, None] // 32 == np.arange(128)[None, :] // 32).astype(np.float32)
_P16 = (np.arange(128)[:, None] // 16 == np.arange(128)[None, :]).astype(np.float32)
_GC = np.zeros((128, 128), np.float32)
for _c in range(CH):
    for _c2 in range(CH):
        if _c // 2 == _c2 // 2:
            _GC[_c, _c2] = 1.0


def _routing_kernel(lg_ref, tri_ref, ex_ref, sc_ref):
    tri = tri_ref[...]
    lane = jax.lax.broadcasted_iota(jnp.int32, (PP, 128), 1)
    for l in range(2):
        lm = lg_ref[l]
        mx = jnp.max(lm, axis=1, keepdims=True)
        ex_exp = jnp.exp(lm - mx)
        probs = ex_exp / jnp.sum(ex_exp, axis=1, keepdims=True)
        gate = jnp.max(probs, axis=1, keepdims=True)
        eidx = jnp.argmax(lm, axis=1)
        rowmask = jax.lax.broadcasted_iota(jnp.int32, (PP, 1), 0) < P
        oh = jnp.where((lane == eidx[:, None]) & rowmask, 1.0, 0.0)
        cnt = jnp.sum(oh, axis=0, keepdims=True)
        ranks = jnp.dot(tri, oh, preferred_element_type=jnp.float32, precision=jax.lax.Precision.HIGHEST)
        rank_p = jnp.sum(ranks * oh, axis=1, keepdims=True) - 1.0
        cnt_p = jnp.sum(cnt * oh, axis=1, keepdims=True)
        bcol = lane.astype(jnp.float32)
        kept = ((cnt_p * bcol + rank_p) < CAP).astype(jnp.float32)
        ex_ref[l] = jnp.broadcast_to(eidx[:, None], (PP, 128))
        sc_ref[l] = gate * kept


def _stem_kernel(xs_ref, wb_ref, gb_ref, out_ref):
    xs = xs_ref[0]
    ymod = jax.lax.broadcasted_iota(jnp.int32, (B * H, 1), 0) % H
    up = jnp.where(ymod == H - 1, 0.0, jnp.roll(xs, -1, axis=0))
    dn = jnp.where(ymod == 0, 0.0, jnp.roll(xs, 1, axis=0))
    bf = jnp.bfloat16
    acc = (jnp.dot(dn.astype(bf), wb_ref[0].astype(bf), preferred_element_type=jnp.float32)
           + jnp.dot(xs.astype(bf), wb_ref[1].astype(bf), preferred_element_type=jnp.float32)
           + jnp.dot(up.astype(bf), wb_ref[2].astype(bf), preferred_element_type=jnp.float32))
    acc = acc * gb_ref[0:1, :] + gb_ref[1:2, :]
    z = acc * jax.nn.sigmoid(acc)
    out_ref[0] = z.reshape(B, HP, PATCH, 128)


def _moe_kernel(ex0_ref, ex1_ref, t_ref, wb_ref, rs0_ref, rs1_ref,
                m1_ref, m1t_ref, g_ref, p16_ref, gb0_ref,
                sums_ref, sumsq_ref, comp1_ref, comp2_ref):
    i = pl.program_id(0)
    m1 = m1_ref[...]
    m1t = m1t_ref[...]
    ymod = jax.lax.broadcasted_iota(jnp.int32, (B * PATCH, 1), 0) % PATCH

    def conv(z, l, e):
        w3 = wb_ref[l, e]
        up = jnp.where(ymod == PATCH - 1, 0.0, jnp.roll(z, -1, axis=0))
        dn = jnp.where(ymod == 0, 0.0, jnp.roll(z, 1, axis=0))
        bf = jnp.bfloat16
        return (jnp.dot(dn.astype(bf), w3[0].astype(bf), preferred_element_type=jnp.float32)
                + jnp.dot(z.astype(bf), w3[1].astype(bf), preferred_element_type=jnp.float32)
                + jnp.dot(up.astype(bf), w3[2].astype(bf), preferred_element_type=jnp.float32))

    def patch_contrib(j):
        x = t_ref[j, :, 0].reshape(B * PATCH, 128)
        e0 = ex0_ref[2 * i + j]
        y = conv(x, 0, e0)
        y = y * jax.nn.sigmoid(y)
        y = y * rs0_ref[j]
        hi = jax.lax.Precision.HIGHEST
        s1 = jnp.dot(m1, y, preferred_element_type=jnp.float32, precision=hi)
        s2 = jnp.dot(m1, y * y, preferred_element_type=jnp.float32, precision=hi)
        tg = jnp.dot(s1, g_ref[...], preferred_element_type=jnp.float32, precision=hi)
        t2g = jnp.dot(s2, g_ref[...], preferred_element_type=jnp.float32, precision=hi)
        mean = tg * (1.0 / 512.0)
        var = t2g * (1.0 / 512.0) - mean * mean
        inv = jax.lax.rsqrt(var + 1e-5)
        gam = gb0_ref[0:1, :]
        bet = gb0_ref[1:2, :]
        a = inv * gam
        c = bet - mean * a
        z = y * jnp.dot(m1t, a, preferred_element_type=jnp.float32, precision=hi) \
            + jnp.dot(m1t, c, preferred_element_type=jnp.float32, precision=hi)
        e1 = ex1_ref[2 * i + j]
        y2 = conv(z, 1, e1)
        y2 = y2 * jax.nn.sigmoid(y2)
        y2 = y2 * rs1_ref[j]
        cs = jnp.dot(jnp.dot(m1, y2, preferred_element_type=jnp.float32, precision=hi),
                     p16_ref[...], preferred_element_type=jnp.float32, precision=hi)
        cs2 = jnp.dot(jnp.dot(m1, y2 * y2, preferred_element_type=jnp.float32, precision=hi),
                      p16_ref[...], preferred_element_type=jnp.float32, precision=hi)
        return cs, cs2

    cs_a, cs2_a = patch_contrib(0)
    cs_b, cs2_b = patch_contrib(1)
    cs = cs_a + cs_b
    cs2 = cs2_a + cs2_b

    @pl.when(i == 0)
    def _():
        sums_ref[...] = jnp.zeros_like(sums_ref)
        sumsq_ref[...] = jnp.zeros_like(sumsq_ref)
        comp1_ref[...] = jnp.zeros_like(comp1_ref)
        comp2_ref[...] = jnp.zeros_like(comp2_ref)

    for inc, acc_ref, comp_ref in ((cs, sums_ref, comp1_ref),
                                   (cs2, sumsq_ref, comp2_ref)):
        yk = inc - comp_ref[...]
        s_old = acc_ref[...]
        t = s_old + yk
        comp_ref[...] = (t - s_old) - yk
        acc_ref[...] = t


def _head_kernel(sums_ref, sumsq_ref, gc_ref, gb1_ref, w1_ref, b1_ref,
                 w2_ref, b2_ref, out_ref):
    hw = float(H * W)
    sums = sums_ref[...]
    sumsq = sumsq_ref[...]
    m = jnp.dot(sums, gc_ref[...], preferred_element_type=jnp.float32, precision=jax.lax.Precision.HIGHEST) * (1.0 / (2.0 * hw))
    ex2 = jnp.dot(sumsq, gc_ref[...], preferred_element_type=jnp.float32, precision=jax.lax.Precision.HIGHEST) * (1.0 / (2.0 * hw))
    var = ex2 - m * m
    inv = jax.lax.rsqrt(var + 1e-5)
    pooled = (sums * (1.0 / hw) - m) * inv * gb1_ref[0:1, :] + gb1_ref[1:2, :]
    h = jnp.dot(pooled, w1_ref[...], preferred_element_type=jnp.float32, precision=jax.lax.Precision.HIGHEST) + b1_ref[...]
    h = jax.nn.gelu(h)
    out_ref[...] = jnp.dot(h, w2_ref[...], preferred_element_type=jnp.float32, precision=jax.lax.Precision.HIGHEST) + b2_ref[...]


def _band_weights(ew):
    xi = np.arange(PATCH)[:, None]
    xo = np.arange(PATCH)[None, :]
    kxm = xi - xo + 1
    valid = jnp.asarray((kxm >= 0) & (kxm <= 2), jnp.float32)
    kxc = jnp.asarray(np.clip(kxm, 0, 2))
    w6 = ew[:, :, :, :, kxc]
    w6 = w6 * valid[None, None, None, None]
    w6 = jnp.transpose(w6, (0, 3, 2, 4, 1, 5))
    return w6.reshape(E, 3, 128, 128)


def kernel(X, stem_w, stem_gamma, stem_beta, gate_w0, gate_w1, exp_w0, exp_w1,
           gn_g0, gn_b0, gn_g1, gn_b1, head_w1, head_b1, head_w2, head_b2):
    f32 = jnp.float32

    gfm = jnp.asarray(_POS).mean(axis=(2, 3))
    gate_feat = jnp.tile(gfm, (B, 1))
    lg0 = (gate_feat @ gate_w0)[:P]
    lg1 = (gate_feat @ gate_w1)[:P]
    lgp = jnp.full((2, PP, 128), -1e30, f32)
    lgp = lgp.at[0, :P, :E].set(lg0).at[1, :P, :E].set(lg1)
    ex_out, sc_out = pl.pallas_call(
        _routing_kernel,
        out_shape=(jax.ShapeDtypeStruct((2, PP, 128), jnp.int32),
                   jax.ShapeDtypeStruct((2, PP, 128), f32)),
    )(lgp, jnp.asarray(_TRI))
    ex0 = ex_out[0, :P, 0]
    ex1 = ex_out[1, :P, 0]
    rs0 = jnp.repeat(sc_out[0, :P, :B], PATCH, axis=1).reshape(P, B * PATCH, 1)
    rs1 = jnp.repeat(sc_out[1, :P, :B], PATCH, axis=1).reshape(P, B * PATCH, 1)

    Xp = jnp.pad(X, ((0, 0), (0, 0), (0, 0), (1, 1)))
    Xs = jnp.stack([Xp[:, :, :, wp * PATCH:wp * PATCH + 18] for wp in range(WP)], 0)
    Xs = Xs.transpose(0, 1, 3, 2, 4).reshape(WP, B * H, CIN * 18)
    Xs = jnp.pad(Xs, ((0, 0), (0, 0), (0, 64 - CIN * 18)))
    cg, cig, dyg, kxg, xog = np.meshgrid(np.arange(CH), np.arange(CIN),
                                         np.arange(3), np.arange(3),
                                         np.arange(PATCH), indexing='ij')
    wsb = jnp.zeros((3, 64, 128), f32).at[
        dyg.ravel(), (cig * 18 + xog + kxg).ravel(), (cg * PATCH + xog).ravel()
    ].set(stem_w[cg.ravel(), cig.ravel(), dyg.ravel(), kxg.ravel()])
    gbs = jnp.stack([jnp.repeat(stem_gamma, PATCH), jnp.repeat(stem_beta, PATCH)])
    T = pl.pallas_call(
        _stem_kernel,
        grid=(WP,),
        in_specs=[
            pl.BlockSpec((1, B * H, 64), lambda w: (w, 0, 0)),
            pl.BlockSpec((3, 64, 128), lambda w: (0, 0, 0)),
            pl.BlockSpec((2, 128), lambda w: (0, 0)),
        ],
        out_specs=pl.BlockSpec((1, B, HP, PATCH, 128), lambda w: (w, 0, 0, 0, 0)),
        out_shape=jax.ShapeDtypeStruct((WP, B, HP, PATCH, 128), f32),
    )(Xs, wsb, gbs)

    wb = jnp.stack([_band_weights(exp_w0), _band_weights(exp_w1)])
    gb0 = jnp.stack([jnp.repeat(gn_g0, PATCH), jnp.repeat(gn_b0, PATCH)])
    grid_spec = pltpu.PrefetchScalarGridSpec(
        num_scalar_prefetch=2,
        grid=(P // 2,),
        in_specs=[
            pl.BlockSpec((2, B, 1, PATCH, 128),
                         lambda i, e0, e1: (i % (WP // 2), 0, i // (WP // 2), 0, 0)),
            pl.BlockSpec((2, E, 3, 128, 128), lambda i, e0, e1: (0, 0, 0, 0, 0)),
            pl.BlockSpec((2, B * PATCH, 1), lambda i, e0, e1: (i, 0, 0)),
            pl.BlockSpec((2, B * PATCH, 1), lambda i, e0, e1: (i, 0, 0)),
            pl.BlockSpec((B, B * PATCH), lambda i, e0, e1: (0, 0)),
            pl.BlockSpec((B * PATCH, B), lambda i, e0, e1: (0, 0)),
            pl.BlockSpec((128, 128), lambda i, e0, e1: (0, 0)),
            pl.BlockSpec((128, 128), lambda i, e0, e1: (0, 0)),
            pl.BlockSpec((2, 128), lambda i, e0, e1: (0, 0)),
        ],
        out_specs=tuple(pl.BlockSpec((B, 128), lambda i, e0, e1: (0, 0))
                        for _ in range(4)),
    )
    sums, sumsq, _, _ = pl.pallas_call(
        _moe_kernel,
        grid_spec=grid_spec,
        out_shape=tuple(jax.ShapeDtypeStruct((B, 128), f32) for _ in range(4)),
    )(ex0, ex1, T, wb, rs0, rs1,
      jnp.asarray(_M1), jnp.asarray(_M1.T), jnp.asarray(_GSUM), jnp.asarray(_P16), gb0)

    gb1 = jnp.zeros((2, 128), f32)
    gb1 = gb1.at[0, :CH].set(gn_g1).at[1, :CH].set(gn_b1)
    w1p = jnp.zeros((128, 128), f32).at[:CH, :4 * CH].set(head_w1)
    b1p = jnp.zeros((1, 128), f32).at[0, :4 * CH].set(head_b1)
    w2p = jnp.zeros((128, 1024), f32).at[:4 * CH, :NUM_CLASSES].set(head_w2)
    b2p = jnp.zeros((1, 1024), f32).at[0, :NUM_CLASSES].set(head_b2)
    out = pl.pallas_call(
        _head_kernel,
        out_shape=jax.ShapeDtypeStruct((B, 1024), f32),
    )(sums, sumsq, jnp.asarray(_GC), gb1, w1p, b1p, w2p, b2p)
    return out[:, :NUM_CLASSES]
